# Initial kernel scaffold; baseline (speedup 1.0000x reference)
import jax
import jax.numpy as jnp
from jax.experimental import pallas as pl


def kernel(user_id, ing_id, recipe_x, edge_ir, edge_ur, user_table, ing_table, W_rl, b_rl, bn_gamma, bn_beta, W_sage_msg, W_sage_root, b_sage, W_gat_src, W_gat_dst, att_src, att_dst, b_gat):
    raise NotImplementedError("write your pallas kernel here")


# speedup vs baseline: 9.1431x; 9.1431x over previous
import functools

import jax
import jax.numpy as jnp
from jax import lax
from jax.experimental import pallas as pl
from jax.experimental.pallas import tpu as pltpu
from jax.experimental.pallas import tpu_sc as plsc

f32 = jnp.float32
i32 = jnp.int32

NU = 50000
NR = 50000
NI = 10000
D = 128
FIN = 20
E1 = 200000
E2 = 400000

NC = 2
NS = 16
NW = NC * NS

BU = 51200
BI = 10240
E1P = 204800
E2P = 409600
NACC1 = 10240
NACC2 = 51200
NT = 51200

CE1 = 200
CE2 = 512
CEN = 256

_CP = pltpu.CompilerParams(use_tc_tiling_on_sc=False)


@functools.lru_cache(maxsize=None)
def _sc_mesh():
    return plsc.VectorSubcoreMesh(core_axis_name="c", subcore_axis_name="s",
                                  num_cores=NC, num_subcores=NS)


def _wid():
    return lax.axis_index("s") * NC + lax.axis_index("c")


@functools.lru_cache(maxsize=None)
def _make_gather(B, C):
    per_w = B // NW
    n_iter = per_w // C

    @functools.partial(
        pl.kernel,
        out_type=jax.ShapeDtypeStruct((B, D), f32),
        mesh=_sc_mesh(),
        compiler_params=_CP,
        scratch_types=[
            pltpu.VMEM((C,), i32),
            pltpu.VMEM((C, D), f32),
            pltpu.SemaphoreType.DMA,
        ],
    )
    def k(table, idx, out, idx_v, rows_v, sem):
        base = _wid() * per_w

        def step(j, carry):
            off = base + j * C
            pltpu.sync_copy(idx.at[pl.ds(off, C)], idx_v)
            pltpu.async_copy(table.at[idx_v], rows_v, sem).wait()
            pltpu.sync_copy(rows_v, out.at[pl.ds(off, C)])
            return carry

        lax.fori_loop(0, n_iter, step, 0)

    return k


@functools.lru_cache(maxsize=None)
def _build_sage_acc():
    return functools.partial(
        pl.kernel,
        out_type=(
            jax.ShapeDtypeStruct((NC * NACC1, D), f32),
            jax.ShapeDtypeStruct((NC * NACC1, 16), f32),
        ),
        mesh=_sc_mesh(),
        compiler_params=_CP,
        scratch_types=[
            pltpu.VMEM((CE1,), i32),
            pltpu.VMEM((CE1,), i32),
            pltpu.VMEM((CE1, D), f32),
            pltpu.VMEM((CE1, 16), f32),
            pltpu.VMEM_SHARED((NACC1, D), f32),
            pltpu.VMEM_SHARED((NACC1, 16), f32),
            pltpu.SemaphoreType.DMA,
        ],
    )(_sage_acc_body)


def _sage_acc_body(tab, src, dst, zrow, zcnt, ones, aggp, cntp,
                   src_v, dst_v, rows_v, ones_v, acc, acc_c, sem):
    cid = lax.axis_index("c")
    sid = lax.axis_index("s")
    wid = sid * NC + cid
    rows_per_tile = NACC1 // NS
    r0 = sid * rows_per_tile

    pltpu.sync_copy(ones, ones_v)
    pltpu.sync_copy(zrow, acc.at[pl.ds(r0, rows_per_tile)])
    pltpu.sync_copy(zcnt, acc_c.at[pl.ds(r0, rows_per_tile)])
    plsc.subcore_barrier()

    base = wid * (E1P // NW)

    def step(j, carry):
        off = base + j * CE1
        pltpu.sync_copy(src.at[pl.ds(off, CE1)], src_v)
        pltpu.sync_copy(dst.at[pl.ds(off, CE1)], dst_v)
        pltpu.async_copy(tab.at[src_v], rows_v, sem).wait()
        pltpu.sync_copy(rows_v, acc.at[dst_v], add=True)
        pltpu.sync_copy(ones_v, acc_c.at[dst_v], add=True)
        return carry

    lax.fori_loop(0, (E1P // NW) // CE1, step, 0)
    plsc.subcore_barrier()

    pltpu.sync_copy(acc.at[pl.ds(r0, rows_per_tile)],
                    aggp.at[pl.ds(cid * NACC1 + r0, rows_per_tile)])
    pltpu.sync_copy(acc_c.at[pl.ds(r0, rows_per_tile)],
                    cntp.at[pl.ds(cid * NACC1 + r0, rows_per_tile)])


@functools.lru_cache(maxsize=None)
def _build_gat_ex():
    return functools.partial(
        pl.kernel,
        out_type=jax.ShapeDtypeStruct((E2P, 16), f32),
        mesh=_sc_mesh(),
        compiler_params=_CP,
        scratch_types=[
            pltpu.VMEM((16,), f32),
            pltpu.VMEM((CE2,), i32),
            pltpu.VMEM((CE2,), i32),
            pltpu.VMEM((CE2, 16), f32),
            pltpu.VMEM((CE2, 16), f32),
            pltpu.VMEM((CE2, 16), f32),
            pltpu.SemaphoreType.DMA,
        ],
    )(_gat_ex_body)


def _gat_ex_body(ash, adh, cmh, s2h, d2h, exh,
                 cm_v, s2_v, d2_v, as_v, ad_v, ex_v, sem):
    pltpu.sync_copy(cmh, cm_v)
    base = _wid() * (E2P // NW)

    def step(j, carry):
        off = base + j * CE2
        pltpu.sync_copy(s2h.at[pl.ds(off, CE2)], s2_v)
        pltpu.sync_copy(d2h.at[pl.ds(off, CE2)], d2_v)
        pltpu.async_copy(ash.at[s2_v], as_v, sem).wait()
        pltpu.async_copy(adh.at[d2_v], ad_v, sem).wait()
        cm = cm_v[...]

        def g(e, c2):
            s = pl.ds(0, 16)
            ev = as_v[e, s] + ad_v[e, s]
            ev = jnp.maximum(ev, 0.2 * ev)
            ex_v[e, s] = jnp.exp(ev - cm)
            return c2

        lax.fori_loop(0, CE2, g, 0)
        pltpu.sync_copy(ex_v, exh.at[pl.ds(off, CE2)])
        return carry

    lax.fori_loop(0, (E2P // NW) // CE2, step, 0)


@functools.lru_cache(maxsize=None)
def _build_gat_num():
    return functools.partial(
        pl.kernel,
        out_type=jax.ShapeDtypeStruct((4 * NACC2, 32), f32),
        mesh=_sc_mesh(),
        compiler_params=_CP,
        scratch_types=[
            pltpu.VMEM((CEN,), i32),
            pltpu.VMEM((CEN,), i32),
            pltpu.VMEM((CEN,), i32),
            pltpu.VMEM((CEN, 16), f32),
            pltpu.VMEM((CEN, 32), f32),
            pltpu.VMEM((CEN, 32), f32),
            pltpu.VMEM_SHARED((NACC2, 32), f32),
            pltpu.SemaphoreType.DMA,
        ],
    )(_gat_num_body)


def _gat_num_body(hsf, s2h, d2h, exh, znum, outn,
                  s2_v, d2_v, idx_v, ex_v, rows_v, sc_v, acc, sem):
    cid = lax.axis_index("c")
    sid = lax.axis_index("s")
    rows_per_tile = NACC2 // NS
    r0 = sid * rows_per_tile
    per_tile = E2P // NS

    for p in range(2):
        fc = 2 * cid + p
        fcb = fc * NU
        pltpu.sync_copy(znum, acc.at[pl.ds(r0, rows_per_tile)])
        plsc.subcore_barrier()

        def step(j, carry):
            off = sid * per_tile + j * CEN
            pltpu.sync_copy(s2h.at[pl.ds(off, CEN)], s2_v)
            pltpu.sync_copy(d2h.at[pl.ds(off, CEN)], d2_v)
            pltpu.sync_copy(exh.at[pl.ds(off, CEN)], ex_v)

            def mkidx(g, c2):
                s = pl.ds(g * 16, 16)
                idx_v[s] = s2_v[s] + fcb
                return c2

            lax.fori_loop(0, CEN // 16, mkidx, 0)
            pltpu.async_copy(hsf.at[idx_v], rows_v, sem).wait()

            def scale(e, c2):
                ex16 = ex_v[e, pl.ds(0, 16)]
                lo = pl.ds(0, 16)
                hi = pl.ds(16, 16)
                sc_v[e, lo] = rows_v[e, lo] * ex16
                sc_v[e, hi] = rows_v[e, hi] * ex16
                return c2

            lax.fori_loop(0, CEN, scale, 0)
            pltpu.sync_copy(sc_v, acc.at[d2_v], add=True)
            return carry

        lax.fori_loop(0, per_tile // CEN, step, 0)
        plsc.subcore_barrier()
        pltpu.sync_copy(acc.at[pl.ds(r0, rows_per_tile)],
                        outn.at[pl.ds(fc * NACC2 + r0, rows_per_tile)])


@functools.lru_cache(maxsize=None)
def _build_gat_den():
    return functools.partial(
        pl.kernel,
        out_type=jax.ShapeDtypeStruct((NC * NACC2, 16), f32),
        mesh=_sc_mesh(),
        compiler_params=_CP,
        scratch_types=[
            pltpu.VMEM((CE2,), i32),
            pltpu.VMEM((CE2, 16), f32),
            pltpu.VMEM_SHARED((NACC2, 16), f32),
        ],
    )(_gat_den_body)


def _gat_den_body(d2h, exh, zden, outd, d2_v, ex_v, acc):
    cid = lax.axis_index("c")
    sid = lax.axis_index("s")
    rows_per_tile = NACC2 // NS
    r0 = sid * rows_per_tile
    per_tile = (E2P // NC) // NS

    pltpu.sync_copy(zden, acc.at[pl.ds(r0, rows_per_tile)])
    plsc.subcore_barrier()
    base = cid * (E2P // NC) + sid * per_tile

    def step(j, carry):
        off = base + j * CE2
        pltpu.sync_copy(d2h.at[pl.ds(off, CE2)], d2_v)
        pltpu.sync_copy(exh.at[pl.ds(off, CE2)], ex_v)
        pltpu.sync_copy(ex_v, acc.at[d2_v], add=True)
        return carry

    lax.fori_loop(0, per_tile // CE2, step, 0)
    plsc.subcore_barrier()
    pltpu.sync_copy(acc.at[pl.ds(r0, rows_per_tile)],
                    outd.at[pl.ds(cid * NACC2 + r0, rows_per_tile)])


BM = 1000
NBLK = NR // BM


def _tc_stats_body(x_ref, xtx_ref, cs_ref):
    i = pl.program_id(0)
    x = x_ref[...]

    @pl.when(i == 0)
    def _():
        xtx_ref[...] = jnp.zeros_like(xtx_ref)
        cs_ref[...] = jnp.zeros_like(cs_ref)

    xtx_ref[...] += lax.dot_general(x, x, (((0,), (0,)), ((), ())),
                                    preferred_element_type=f32)
    cs_ref[...] += jnp.sum(x, axis=0, keepdims=True)


def _tc_stats(x):
    return pl.pallas_call(
        _tc_stats_body,
        grid=(NBLK,),
        in_specs=[pl.BlockSpec((BM, FIN), lambda i: (i, 0))],
        out_specs=[
            pl.BlockSpec((FIN, FIN), lambda i: (0, 0)),
            pl.BlockSpec((1, FIN), lambda i: (0, 0)),
        ],
        out_shape=[
            jax.ShapeDtypeStruct((FIN, FIN), f32),
            jax.ShapeDtypeStruct((1, FIN), f32),
        ],
    )(x)


def _tc_user_body(u_ref, w_ref, att_ref, hs_ref, as_ref, mx_ref):
    i = pl.program_id(0)
    u = u_ref[...]
    n = jnp.sqrt(jnp.sum(u * u, axis=1, keepdims=True))
    u = u * jnp.minimum(1.0, 1.0 / (n + 1e-7))
    hs = jnp.dot(u, w_ref[...], preferred_element_type=f32)
    a_s = jnp.sum(hs * att_ref[...], axis=1, keepdims=True)
    for c in range(4):
        hs_ref[c] = hs[:, c * 32:(c + 1) * 32]
    as_ref[...] = jnp.broadcast_to(a_s, (BM, 16))

    @pl.when(i == 0)
    def _():
        mx_ref[...] = jnp.full((1, 1), -1e30, f32)

    mx_ref[...] = jnp.maximum(mx_ref[...], jnp.full((1, 1), jnp.max(a_s), f32))


def _tc_user(u_raw, w, att):
    return pl.pallas_call(
        _tc_user_body,
        grid=(NBLK,),
        in_specs=[
            pl.BlockSpec((BM, D), lambda i: (i, 0)),
            pl.BlockSpec((D, D), lambda i: (0, 0)),
            pl.BlockSpec((1, D), lambda i: (0, 0)),
        ],
        out_specs=[
            pl.BlockSpec((4, BM, 32), lambda i: (0, i, 0)),
            pl.BlockSpec((BM, 16), lambda i: (i, 0)),
            pl.BlockSpec((1, 1), lambda i: (0, 0)),
        ],
        out_shape=[
            jax.ShapeDtypeStruct((4, NU, 32), f32),
            jax.ShapeDtypeStruct((NU, 16), f32),
            jax.ShapeDtypeStruct((1, 1), f32),
        ],
    )(u_raw, w, att)


def _tc_ad_body(x_ref, xtx_ref, csc_ref, wrl_ref, wrlt_ref, brl_ref, gam_ref,
                bet_ref, wmsg_ref, wroot_ref, bsage_ref, wdst_ref, attd_ref,
                p_ref, cnt_ref, ad_ref, mx_ref):
    i = pl.program_id(0)
    wrlt = wrlt_ref[...]
    w = jnp.dot(wdst_ref[...], attd_ref[...],
                preferred_element_type=f32)
    w2 = w + jnp.dot(wroot_ref[...], w, preferred_element_type=f32)
    w3 = jnp.dot(wmsg_ref[...], w, preferred_element_type=f32)
    brl = brl_ref[...]
    n = float(NR)
    m0 = jnp.dot(wrlt, csc_ref[...], preferred_element_type=f32)
    mu = m0 / n + brl
    tt = jnp.dot(wrlt, xtx_ref[...], preferred_element_type=f32)
    sq = jnp.sum(wrlt * tt, axis=1, keepdims=True)
    ex2 = (sq + 2.0 * brl * m0) / n + brl * brl
    var = ex2 - mu * mu
    s = lax.rsqrt(var + 1e-5)
    q = w2 * gam_ref[...] * s
    const = (jnp.sum((brl - mu) * q) + jnp.sum(bet_ref[...] * w2)
             + jnp.sum(bsage_ref[...] * w))
    vx = jnp.dot(wrl_ref[...], q, preferred_element_type=f32)
    ad = jnp.dot(x_ref[...], vx, preferred_element_type=f32) + const
    p01 = p_ref[0] + p_ref[1]
    dv = jnp.dot(p01, w3, preferred_element_type=f32)
    cn = jnp.maximum(cnt_ref[0, :, 0:1] + cnt_ref[1, :, 0:1], 1.0)
    ad = ad + jnp.where(i < NI // BM, dv / cn, 0.0)
    ad_ref[...] = jnp.broadcast_to(ad, (BM, 16))

    @pl.when(i == 0)
    def _():
        mx_ref[...] = jnp.full((1, 1), -1e30, f32)

    mx_ref[...] = jnp.maximum(mx_ref[...], jnp.full((1, 1), jnp.max(ad), f32))


def _tc_ad(x, xtx, csc, wrl, wrlt, brl, gam, bet, wmsg, wroot, bsage, wdst,
           attd, p, cnt):
    clamp = NI // BM - 1

    return pl.pallas_call(
        _tc_ad_body,
        grid=(NBLK,),
        in_specs=[
            pl.BlockSpec((BM, FIN), lambda i: (i, 0)),
            pl.BlockSpec((FIN, FIN), lambda i: (0, 0)),
            pl.BlockSpec((FIN, 1), lambda i: (0, 0)),
            pl.BlockSpec((FIN, D), lambda i: (0, 0)),
            pl.BlockSpec((D, FIN), lambda i: (0, 0)),
            pl.BlockSpec((D, 1), lambda i: (0, 0)),
            pl.BlockSpec((D, 1), lambda i: (0, 0)),
            pl.BlockSpec((D, 1), lambda i: (0, 0)),
            pl.BlockSpec((D, D), lambda i: (0, 0)),
            pl.BlockSpec((D, D), lambda i: (0, 0)),
            pl.BlockSpec((D, 1), lambda i: (0, 0)),
            pl.BlockSpec((D, D), lambda i: (0, 0)),
            pl.BlockSpec((D, 1), lambda i: (0, 0)),
            pl.BlockSpec((2, BM, D), lambda i: (0, jnp.minimum(i, clamp), 0)),
            pl.BlockSpec((2, BM, 16), lambda i: (0, jnp.minimum(i, clamp), 0)),
        ],
        out_specs=[
            pl.BlockSpec((BM, 16), lambda i: (i, 0)),
            pl.BlockSpec((1, 1), lambda i: (0, 0)),
        ],
        out_shape=[
            jax.ShapeDtypeStruct((NR, 16), f32),
            jax.ShapeDtypeStruct((1, 1), f32),
        ],
    )(x, xtx, csc, wrl, wrlt, brl, gam, bet, wmsg, wroot, bsage, wdst, attd,
      p, cnt)


def _tc_final_body(num_ref, den_ref, bg_ref, out_ref):
    d = den_ref[0] + den_ref[1]
    inv = 1.0 / (d[:, 0:1] + 1e-16)
    for c in range(4):
        out_ref[:, c * 32:(c + 1) * 32] = (
            num_ref[c] * inv + bg_ref[:, c * 32:(c + 1) * 32])


def _tc_final(num, den, bg):
    return pl.pallas_call(
        _tc_final_body,
        grid=(NBLK,),
        in_specs=[
            pl.BlockSpec((4, BM, 32), lambda i: (0, i, 0)),
            pl.BlockSpec((2, BM, 16), lambda i: (0, i, 0)),
            pl.BlockSpec((1, D), lambda i: (0, 0)),
        ],
        out_specs=pl.BlockSpec((BM, D), lambda i: (i, 0)),
        out_shape=jax.ShapeDtypeStruct((NR, D), f32),
    )(num, den, bg)


def kernel(user_id, ing_id, recipe_x, edge_ir, edge_ur, user_table, ing_table,
           W_rl, b_rl, bn_gamma, bn_beta, W_sage_msg, W_sage_root, b_sage,
           W_gat_src, W_gat_dst, att_src, att_dst, b_gat):
    user_id = user_id.astype(i32)
    ing_id = ing_id.astype(i32)

    uid_p = jnp.concatenate([user_id, jnp.arange(BU - NU, dtype=i32) % NU])
    u_raw = _make_gather(BU, 200)(user_table, uid_p)
    iid_p = jnp.concatenate([ing_id, jnp.arange(BI - NI, dtype=i32) % NI])
    ing_perm = _make_gather(BI, 160)(ing_table, iid_p)

    n1 = E1P - E1
    s1 = jnp.concatenate([edge_ir[0].astype(i32),
                          jnp.arange(n1, dtype=i32) % NI])
    d1 = jnp.concatenate([edge_ir[1].astype(i32),
                          NI + jnp.arange(n1, dtype=i32) % (NACC1 - NI)])
    zrow = jnp.zeros((NACC1 // NS, D), f32)
    zc16 = jnp.zeros((NACC1 // NS, 16), f32)
    ones = jnp.ones((CE1, 16), f32)
    aggp, cntp = _build_sage_acc()(ing_perm, s1, d1, zrow, zc16, ones)

    xtx, cs = _tc_stats(recipe_x)
    hs_chunks, as16, max_as = _tc_user(u_raw[:NU], W_gat_src,
                                       att_src.reshape(1, D))
    ad16, max_ad = _tc_ad(recipe_x, xtx, cs.reshape(FIN, 1), W_rl, W_rl.T,
                          b_rl.reshape(D, 1), bn_gamma.reshape(D, 1),
                          bn_beta.reshape(D, 1), W_sage_msg, W_sage_root,
                          b_sage.reshape(D, 1), W_gat_dst,
                          att_dst.reshape(D, 1),
                          aggp.reshape(NC, NACC1, D),
                          cntp.reshape(NC, NACC1, 16))

    m = max_as[0, 0] + max_ad[0, 0]
    cmax = jnp.maximum(m, 0.2 * m)
    cm_v = jnp.full((16,), cmax, f32)

    n2 = E2P - E2
    s2 = jnp.concatenate([edge_ur[0].astype(i32),
                          jnp.arange(n2, dtype=i32) % NU])
    d2 = jnp.concatenate([edge_ur[1].astype(i32),
                          NR + jnp.arange(n2, dtype=i32) % (NACC2 - NR)])
    asp = jnp.concatenate([as16, jnp.zeros((NT - NU, 16), f32)])
    adp = jnp.concatenate([ad16, jnp.zeros((NT - NR, 16), f32)])

    exb = _build_gat_ex()(asp, adp, cm_v, s2, d2)

    hs_flat = hs_chunks.reshape(4 * NU, 32)
    znum = jnp.zeros((NACC2 // NS, 32), f32)
    zden = jnp.zeros((NACC2 // NS, 16), f32)
    nums = _build_gat_num()(hs_flat, s2, d2, exb, znum)
    dens = _build_gat_den()(d2, exb, zden)

    return _tc_final(nums.reshape(4, NACC2, 32),
                     dens.reshape(NC, NACC2, 16),
                     b_gat.reshape(1, D))
